# Initial kernel scaffold; baseline (speedup 1.0000x reference)
import jax
import jax.numpy as jnp
from jax.experimental import pallas as pl


def kernel(x, edge_index, edge_attr, adj, mask, W0in, b0in, Wp1, bp1, W1in, b1in, Wp2, bp2, W2in, b2in, W1out, b1out, W0out, b0out):
    raise NotImplementedError("write your pallas kernel here")


# speedup vs baseline: 33.4735x; 33.4735x over previous
import functools

import jax
import jax.numpy as jnp
from jax import lax
from jax.experimental import pallas as pl
from jax.experimental.pallas import tpu as pltpu
from jax.experimental.pallas import tpu_sc as plsc

N = 4096
E = 65536
F_IN = 128
HID = 64
C1 = 512
C2 = 64
NCLS = 16

NC = 2
NS = 16
NW = NC * NS
L = 16

EPW = E // NW
KCH = 128
NCHUNK = EPW // KCH
RPT = N // NS


def _sc_mesh():
  return plsc.VectorSubcoreMesh(
      core_axis_name="c", subcore_axis_name="s",
      num_cores=NC, num_subcores=NS)


def _edge_scatter_sc(src3d, dst3d, ew2d, v, zeros, d):
  @functools.partial(
      pl.kernel,
      out_type=jax.ShapeDtypeStruct((NC, N, d), jnp.float32),
      mesh=_sc_mesh(),
      scratch_types=[
          pltpu.VMEM((NCHUNK, KCH), jnp.int32),
          pltpu.VMEM((NCHUNK, KCH), jnp.int32),
          pltpu.VMEM((EPW,), jnp.float32),
          pltpu.VMEM((KCH, d), jnp.float32),
          pltpu.VMEM_SHARED((N, d), jnp.float32),
          pltpu.SemaphoreType.DMA,
      ],
      compiler_params=pltpu.CompilerParams(use_tc_tiling_on_sc=False),
  )
  def k(src_hbm, dst_hbm, ew_hbm, v_hbm, z_hbm, out_hbm,
        src_v, dst_v, ew_v, rows_v, acc_sh, sem):
    c = lax.axis_index("c")
    s = lax.axis_index("s")
    wid = s * NC + c
    pltpu.sync_copy(z_hbm.at[pl.ds(s * RPT, RPT)],
                    acc_sh.at[pl.ds(s * RPT, RPT)])
    pltpu.sync_copy(src_hbm.at[wid], src_v)
    pltpu.sync_copy(dst_hbm.at[wid], dst_v)
    pltpu.sync_copy(ew_hbm.at[wid], ew_v)
    plsc.subcore_barrier()

    @pl.loop(0, NCHUNK)
    def _chunk(j):
      pltpu.async_copy(v_hbm.at[src_v.at[j]], rows_v, sem).wait()

      @pl.loop(0, KCH // L)
      def _scale(gi):
        wv = ew_v[pl.ds(j * KCH + gi * L, L)]
        for i in range(L):
          w = wv[i]
          e = gi * L + i
          for q in range(d // L):
            sl = pl.ds(q * L, L)
            rows_v[e, sl] = rows_v[e, sl] * w

      pltpu.sync_copy(rows_v, acc_sh.at[dst_v.at[j]], add=True)

    plsc.subcore_barrier()
    pltpu.sync_copy(acc_sh.at[pl.ds(s * RPT, RPT)],
                    out_hbm.at[c, pl.ds(s * RPT, RPT)])

  return k(src3d, dst3d, ew2d, v, zeros)


def _prep_tc(deg_parts, x):
  def body(degp_ref, x_ref, dinv_ref, v1_ref):
    deg = degp_ref[0, :, 0:1] + degp_ref[1, :, 0:1]
    dinv = lax.rsqrt(deg + 1.0)
    dinv_ref[...] = dinv
    v1_ref[...] = x_ref[...] * dinv

  return pl.pallas_call(
      body,
      out_shape=(jax.ShapeDtypeStruct((N, 1), jnp.float32),
                 jax.ShapeDtypeStruct((N, F_IN), jnp.float32)),
  )(deg_parts, x)


def _conv_finish_tc(s_parts, v, dinv, w, b, d_in, d_out):
  def body(sp_ref, v_ref, dinv_ref, w_ref, b_ref, x_ref, vn_ref):
    u = (sp_ref[0] + sp_ref[1] + v_ref[...]) * dinv_ref[...]
    h = jnp.dot(u, w_ref[...], preferred_element_type=jnp.float32)
    h = jnp.maximum(h + b_ref[...], 0.0)
    x_ref[...] = h
    vn_ref[...] = h * dinv_ref[...]

  return pl.pallas_call(
      body,
      out_shape=(jax.ShapeDtypeStruct((N, d_out), jnp.float32),
                 jax.ShapeDtypeStruct((N, d_out), jnp.float32)),
  )(s_parts, v, dinv, w, b)


BR = 512
GR = N // BR


def _s1_softmax_tc(s_parts, v2, dinv, wp1, bp1):
  def body(sp_ref, v_ref, dinv_ref, w_ref, b_ref, s1_ref, sm_ref, ent_ref):
    u = (sp_ref[0] + sp_ref[1] + v_ref[...]) * dinv_ref[...]
    s1b = jnp.dot(u, w_ref[...], preferred_element_type=jnp.float32)
    s1b = jnp.maximum(s1b + b_ref[...], 0.0)
    s1_ref[...] = s1b
    m = jnp.max(s1b, axis=1, keepdims=True)
    ex = jnp.exp(s1b - m)
    sm = ex / jnp.sum(ex, axis=1, keepdims=True)
    sm_ref[...] = sm
    entb = jnp.sum(-sm * jnp.log(sm + 1e-15))

    @pl.when(pl.program_id(0) == 0)
    def _():
      ent_ref[...] = jnp.zeros((1, 1), jnp.float32)

    ent_ref[...] = ent_ref[...] + entb

  return pl.pallas_call(
      body,
      grid=(GR,),
      in_specs=[
          pl.BlockSpec((2, BR, HID), lambda i: (0, i, 0)),
          pl.BlockSpec((BR, HID), lambda i: (i, 0)),
          pl.BlockSpec((BR, 1), lambda i: (i, 0)),
          pl.BlockSpec((HID, C1), lambda i: (0, 0)),
          pl.BlockSpec((1, C1), lambda i: (0, 0)),
      ],
      out_specs=(
          pl.BlockSpec((BR, C1), lambda i: (i, 0)),
          pl.BlockSpec((BR, C1), lambda i: (i, 0)),
          pl.BlockSpec((1, 1), lambda i: (0, 0)),
      ),
      out_shape=(jax.ShapeDtypeStruct((N, C1), jnp.float32),
                 jax.ShapeDtypeStruct((N, C1), jnp.float32),
                 jax.ShapeDtypeStruct((1, 1), jnp.float32)),
  )(s_parts, v2, dinv, wp1, bp1)


def _adj_sweep_tc(adj, ssm, x0):
  def body(adj_ref, sb_ref, sf_ref, x0_ref, adj1_ref, x1_ref, g_ref, q_ref):
    i = pl.program_id(0)
    tmp = jnp.dot(adj_ref[...], sf_ref[...],
                  preferred_element_type=jnp.float32)
    sb = sb_ref[...]
    cn = (((0,), (0,)), ((), ()))
    a1 = lax.dot_general(sb, tmp, cn, preferred_element_type=jnp.float32)
    x1 = lax.dot_general(sb, x0_ref[...], cn,
                         preferred_element_type=jnp.float32)
    g = lax.dot_general(sb, sb, cn, preferred_element_type=jnp.float32)
    q = jnp.sum(adj_ref[...] * adj_ref[...]).reshape(1, 1)

    @pl.when(i == 0)
    def _():
      adj1_ref[...] = jnp.zeros_like(adj1_ref)
      x1_ref[...] = jnp.zeros_like(x1_ref)
      g_ref[...] = jnp.zeros_like(g_ref)
      q_ref[...] = jnp.zeros_like(q_ref)

    adj1_ref[...] = adj1_ref[...] + a1
    x1_ref[...] = x1_ref[...] + x1
    g_ref[...] = g_ref[...] + g
    q_ref[...] = q_ref[...] + q

  return pl.pallas_call(
      body,
      grid=(GR,),
      in_specs=[
          pl.BlockSpec((BR, N), lambda i: (i, 0)),
          pl.BlockSpec((BR, C1), lambda i: (i, 0)),
          pl.BlockSpec((N, C1), lambda i: (0, 0)),
          pl.BlockSpec((BR, HID), lambda i: (i, 0)),
      ],
      out_specs=(
          pl.BlockSpec((C1, C1), lambda i: (0, 0)),
          pl.BlockSpec((C1, HID), lambda i: (0, 0)),
          pl.BlockSpec((C1, C1), lambda i: (0, 0)),
          pl.BlockSpec((1, 1), lambda i: (0, 0)),
      ),
      out_shape=(jax.ShapeDtypeStruct((C1, C1), jnp.float32),
                 jax.ShapeDtypeStruct((C1, HID), jnp.float32),
                 jax.ShapeDtypeStruct((C1, C1), jnp.float32),
                 jax.ShapeDtypeStruct((1, 1), jnp.float32)),
  )(adj, ssm, ssm, x0)


def _mid_levels_tc(adj1, x1, g, adjsq, ent1,
                   wp2, bp2, w1in, b1in, w2in, b2in, w1out, b1out):
  def body(adj1_ref, x1_ref, g_ref, q_ref, e1_ref,
           wp2_ref, bp2_ref, w1in_ref, b1in_ref, w2in_ref, b2in_ref,
           w1out_ref, b1out_ref,
           adj2_ref, x1out_ref, loss_ref):
    cn = (((0,), (0,)), ((), ()))
    f32 = jnp.float32
    adj1 = adj1_ref[...]

    eye1 = (lax.broadcasted_iota(jnp.int32, (C1, C1), 0)
            == lax.broadcasted_iota(jnp.int32, (C1, C1), 1)).astype(f32)
    tr1 = jnp.sum(adj1 * eye1)
    gsq = jnp.sum(g_ref[...] * g_ref[...])
    l1 = jnp.sqrt(jnp.maximum(q_ref[0, 0] - 2.0 * tr1 + gsq, 0.0))
    l1 = l1 / (float(N) * float(N))
    e1 = e1_ref[0, 0] / float(N)

    ones1 = jnp.ones((C1, 1), f32)
    deg1 = lax.dot_general(adj1, ones1, cn,
                           preferred_element_type=f32) + 1.0
    dinv1 = lax.rsqrt(deg1)
    v = x1_ref[...] * dinv1
    t = lax.dot_general(adj1, v, cn, preferred_element_type=f32) + v
    qn = t * dinv1
    x1_ = jnp.maximum(
        jnp.dot(qn, w1in_ref[...], preferred_element_type=f32)
        + b1in_ref[...], 0.0)
    vs = x1_ * dinv1
    ts = lax.dot_general(adj1, vs, cn, preferred_element_type=f32) + vs
    s2r = jnp.maximum(
        jnp.dot(ts * dinv1, wp2_ref[...], preferred_element_type=f32)
        + bp2_ref[...], 0.0)

    m = jnp.max(s2r, axis=1, keepdims=True)
    ex = jnp.exp(s2r - m)
    s2 = ex / jnp.sum(ex, axis=1, keepdims=True)
    e2 = jnp.sum(-s2 * jnp.log(s2 + 1e-15)) / float(C1)
    x2 = lax.dot_general(s2, x1_, cn, preferred_element_type=f32)
    a2l = lax.dot_general(s2, adj1, cn, preferred_element_type=f32)
    adj2 = jnp.dot(a2l, s2, preferred_element_type=f32)
    adj2_ref[...] = adj2

    eye2 = (lax.broadcasted_iota(jnp.int32, (C2, C2), 0)
            == lax.broadcasted_iota(jnp.int32, (C2, C2), 1)).astype(f32)
    tr2 = jnp.sum(adj2 * eye2)
    g2 = lax.dot_general(s2, s2, cn, preferred_element_type=f32)
    g2sq = jnp.sum(g2 * g2)
    a1sq = jnp.sum(adj1 * adj1)
    l2 = jnp.sqrt(jnp.maximum(a1sq - 2.0 * tr2 + g2sq, 0.0))
    l2 = l2 / (float(C1) * float(C1))

    ones2 = jnp.ones((C2, 1), f32)
    deg2 = lax.dot_general(adj2, ones2, cn,
                           preferred_element_type=f32) + 1.0
    dinv2 = lax.rsqrt(deg2)
    v2l = x2 * dinv2
    t2 = lax.dot_general(adj2, v2l, cn, preferred_element_type=f32) + v2l
    x2out = jnp.maximum(
        jnp.dot(t2 * dinv2, w2in_ref[...], preferred_element_type=f32)
        + b2in_ref[...], 0.0)

    up2 = jnp.dot(s2r, x2out, preferred_element_type=f32)
    cat1 = jnp.concatenate([x1_, up2], axis=1)
    vc = cat1 * dinv1
    t3 = lax.dot_general(adj1, vc, cn, preferred_element_type=f32) + vc
    x1out = jnp.maximum(
        jnp.dot(t3 * dinv1, w1out_ref[...], preferred_element_type=f32)
        + b1out_ref[...], 0.0)
    x1out_ref[...] = x1out

    loss_ref[...] = (l1 + e1 + l2 + e2).reshape(1, 1)

  return pl.pallas_call(
      body,
      out_shape=(jax.ShapeDtypeStruct((C2, C2), jnp.float32),
                 jax.ShapeDtypeStruct((C1, HID), jnp.float32),
                 jax.ShapeDtypeStruct((1, 1), jnp.float32)),
  )(adj1, x1, g, adjsq, ent1,
    wp2, bp2, w1in, b1in, w2in, b2in, w1out, b1out)


def _upsample_cat_tc(ssm, x1out, x0, dinv):
  def body(sm_ref, xo_ref, x0_ref, dinv_ref, v3_ref):
    up = jnp.dot(sm_ref[...], xo_ref[...], preferred_element_type=jnp.float32)
    cat = jnp.concatenate([x0_ref[...], up], axis=1)
    v3_ref[...] = cat * dinv_ref[...]

  return pl.pallas_call(
      body,
      grid=(GR,),
      in_specs=[
          pl.BlockSpec((BR, C1), lambda i: (i, 0)),
          pl.BlockSpec((C1, HID), lambda i: (0, 0)),
          pl.BlockSpec((BR, HID), lambda i: (i, 0)),
          pl.BlockSpec((BR, 1), lambda i: (i, 0)),
      ],
      out_specs=pl.BlockSpec((BR, 2 * HID), lambda i: (i, 0)),
      out_shape=jax.ShapeDtypeStruct((N, 2 * HID), jnp.float32),
  )(ssm, x1out, x0, dinv)


def _final_tc(s_parts, v3, dinv, w0out, b0out):
  def body(sp_ref, v_ref, dinv_ref, w_ref, b_ref, p_ref):
    u = (sp_ref[0] + sp_ref[1] + v_ref[...]) * dinv_ref[...]
    z = jnp.dot(u, w_ref[...], preferred_element_type=jnp.float32)
    z = jnp.maximum(z + b_ref[...], 0.0)
    m = jnp.max(z, axis=1, keepdims=True)
    lse = jnp.log(jnp.sum(jnp.exp(z - m), axis=1, keepdims=True))
    p_ref[...] = z - m - lse

  return pl.pallas_call(
      body,
      grid=(GR,),
      in_specs=[
          pl.BlockSpec((2, BR, 2 * HID), lambda i: (0, i, 0)),
          pl.BlockSpec((BR, 2 * HID), lambda i: (i, 0)),
          pl.BlockSpec((BR, 1), lambda i: (i, 0)),
          pl.BlockSpec((2 * HID, NCLS), lambda i: (0, 0)),
          pl.BlockSpec((1, NCLS), lambda i: (0, 0)),
      ],
      out_specs=pl.BlockSpec((BR, NCLS), lambda i: (i, 0)),
      out_shape=jax.ShapeDtypeStruct((N, NCLS), jnp.float32),
  )(s_parts, v3, dinv, w0out, b0out)


def kernel(x, edge_index, edge_attr, adj, mask, W0in, b0in, Wp1, bp1,
           W1in, b1in, Wp2, bp2, W2in, b2in, W1out, b1out, W0out, b0out):
  del mask
  src = edge_index[0].astype(jnp.int32)
  dst = edge_index[1].astype(jnp.int32)
  ew = edge_attr.astype(jnp.float32)

  src3d = src.reshape(NW, NCHUNK, KCH)
  dst3d = dst.reshape(NW, NCHUNK, KCH)
  ew2d = ew.reshape(NW, EPW)
  zeros128 = jnp.zeros((N, 2 * HID), jnp.float32)
  zeros64 = jnp.zeros((N, HID), jnp.float32)
  zeros16 = jnp.zeros((N, L), jnp.float32)
  ones16 = jnp.ones((N, L), jnp.float32)

  deg_parts = _edge_scatter_sc(src3d, dst3d, ew2d, ones16, zeros16, L)
  dinv, v1 = _prep_tc(deg_parts, x)

  s0p = _edge_scatter_sc(src3d, dst3d, ew2d, v1, zeros128, F_IN)
  x0, v2 = _conv_finish_tc(s0p, v1, dinv, W0in, b0in.reshape(1, HID),
                           F_IN, HID)

  s1p = _edge_scatter_sc(src3d, dst3d, ew2d, v2, zeros64, HID)
  s1, ssm, ent1 = _s1_softmax_tc(s1p, v2, dinv, Wp1, bp1.reshape(1, C1))

  adj1, x1, g, adjsq = _adj_sweep_tc(adj, ssm, x0)

  adj2, x1out, loss = _mid_levels_tc(
      adj1, x1, g, adjsq, ent1,
      Wp2, bp2.reshape(1, C2), W1in, b1in.reshape(1, HID),
      W2in, b2in.reshape(1, HID), W1out, b1out.reshape(1, HID))

  v3 = _upsample_cat_tc(s1, x1out, x0, dinv)
  s2p = _edge_scatter_sc(src3d, dst3d, ew2d, v3, zeros128, 2 * HID)
  prediction = _final_tc(s2p, v3, dinv, W0out, b0out.reshape(1, NCLS))

  r1 = jnp.repeat(jnp.arange(C1, dtype=jnp.int32), C1)
  c1 = jnp.tile(jnp.arange(C1, dtype=jnp.int32), C1)
  ei1 = jnp.stack([r1, c1])
  ew1 = adj1.reshape(-1)
  r2 = jnp.repeat(jnp.arange(C2, dtype=jnp.int32), C2)
  c2 = jnp.tile(jnp.arange(C2, dtype=jnp.int32), C2)
  ei2 = jnp.stack([r2, c2])
  ew2 = adj2.reshape(-1)

  return (prediction, s1, loss[0, 0], adj1, ei1, ew1, ei2, ew2)
